# baseline (speedup 1.0000x reference)
import functools
import math

import jax
import jax.numpy as jnp
from jax import lax
from jax.experimental import pallas as pl
from jax.experimental.pallas import tpu as pltpu
from jax.experimental.pallas import tpu_sc as plsc

D_MODEL = 64
SCALE = math.sqrt(D_MODEL)
LANES = 16
NUM_CORES = 2
NUM_SUBCORES = 16
NUM_WORKERS = NUM_CORES * NUM_SUBCORES

CSUB = 128
KSUB = 8
BLOCK = KSUB * CSUB


@functools.lru_cache(maxsize=None)
def _build(B):
    b_per_w = B // NUM_WORKERS
    nblocks = b_per_w // BLOCK
    rows_per_w = b_per_w // CSUB

    mesh = plsc.VectorSubcoreMesh(
        core_axis_name="c", subcore_axis_name="s",
        num_cores=NUM_CORES, num_subcores=NUM_SUBCORES)

    @functools.partial(
        pl.kernel,
        mesh=mesh,
        out_type=jax.ShapeDtypeStruct((B, D_MODEL), jnp.float32),
        scratch_types=[
            pltpu.VMEM((KSUB, CSUB), jnp.int32),
            pltpu.VMEM((BLOCK, D_MODEL), jnp.float32),
            pltpu.SemaphoreType.DMA,
        ],
        compiler_params=pltpu.CompilerParams(use_tc_tiling_on_sc=False),
    )
    def emb(x_hbm, table_hbm, out_hbm, idx_v, rows_v, sem):
        wid = lax.axis_index("s") * NUM_CORES + lax.axis_index("c")
        row_base = wid * rows_per_w

        def block_body(bi, carry):
            row_off = row_base + bi * KSUB
            flat_off = row_off * CSUB
            pltpu.sync_copy(x_hbm.at[pl.ds(row_off, KSUB)], idx_v)
            copies = []
            for j in range(KSUB):
                copies.append(pltpu.async_copy(
                    table_hbm.at[idx_v.at[j]],
                    rows_v.at[pl.ds(j * CSUB, CSUB)],
                    sem))
            for c in copies:
                c.wait()

            def scale_row(r, c2):
                for c in range(D_MODEL // LANES):
                    sl = pl.ds(c * LANES, LANES)
                    rows_v[r, sl] = rows_v[r, sl] * SCALE
                return c2

            lax.fori_loop(0, BLOCK, scale_row, 0, unroll=4)

            pltpu.sync_copy(rows_v, out_hbm.at[pl.ds(flat_off, BLOCK)])
            return carry

        lax.fori_loop(0, nblocks, block_body, 0)

    return emb


def kernel(x, table):
    S0, S1 = x.shape
    B = S0 * S1
    xr = x.reshape(B // CSUB, CSUB).astype(jnp.int32)
    out = _build(B)(xr, table)
    return out.reshape(S0, S1, D_MODEL)
